# baseline (speedup 1.0000x reference)
import jax
import jax.numpy as jnp
from jax.experimental import pallas as pl
from jax.experimental.pallas import tpu as pltpu

_LANE = 128
_TN = 8192


def _round_up(v, m):
    return (v + m - 1) // m * m


def _maha_kernel(x0_ref, x1_ref, means_ref, alpha_ref, out_ref):
    bf16 = jnp.bfloat16
    f32 = jnp.float32
    dh = x0_ref.shape[1]
    mb = means_ref[...]
    abb = alpha_ref[...].astype(bf16)
    mbb = mb.astype(bf16)
    am = jnp.dot(abb, mbb, preferred_element_type=f32)
    atm = jax.lax.dot_general(abb, mbb, (((0,), (0,)), ((), ())),
                              preferred_element_type=f32)
    t4 = jnp.sum(mb * am, axis=0, keepdims=True)
    m2 = (am + atm).astype(bf16)

    x0 = x0_ref[...]
    x1 = x1_ref[...]
    xb0 = x0.astype(bf16)
    xb1 = x1.astype(bf16)
    term23 = (jnp.dot(xb0, m2[:dh, :], preferred_element_type=f32) +
              jnp.dot(xb1, m2[dh:, :], preferred_element_type=f32))
    xa = (jnp.dot(xb0, abb[:dh, :], preferred_element_type=f32) +
          jnp.dot(xb1, abb[dh:, :], preferred_element_type=f32))
    term1 = (jnp.sum(xa[:, :dh] * x0, axis=1, keepdims=True) +
             jnp.sum(xa[:, dh:] * x1, axis=1, keepdims=True))
    qmin = term1 + jnp.min(t4 - term23, axis=1, keepdims=True)
    packed = jnp.broadcast_to(qmin, (qmin.shape[0], _LANE))
    out_ref[...] = packed.T[:1, :]


def kernel(x, means, alpha):
    n, d = x.shape
    d_m, c = means.shape
    assert d == d_m and alpha.shape == (d, d)
    assert d % (2 * _LANE) == 0

    f32 = jnp.float32
    x = x.astype(f32)
    means = means.astype(f32)
    alpha = alpha.astype(f32)

    tn = min(_TN, _round_up(n, _LANE))
    n_pad = _round_up(n, tn)
    num_tiles = n_pad // tn
    x_p = x if n_pad == n else jnp.zeros((n_pad, d), f32).at[:n, :].set(x)
    dh = d // 2

    out = pl.pallas_call(
        _maha_kernel,
        out_shape=jax.ShapeDtypeStruct((num_tiles, 1, tn), f32),
        grid=(num_tiles,),
        in_specs=[
            pl.BlockSpec((tn, dh), lambda i: (i, 0)),
            pl.BlockSpec((tn, dh), lambda i: (i, 1)),
            pl.BlockSpec((d, c), lambda i: (0, 0),
                         pipeline_mode=pl.Buffered(1)),
            pl.BlockSpec((d, d), lambda i: (0, 0),
                         pipeline_mode=pl.Buffered(1)),
        ],
        out_specs=pl.BlockSpec((None, 1, tn), lambda i: (i, 0, 0)),
        compiler_params=pltpu.CompilerParams(
            dimension_semantics=("parallel",),
            vmem_limit_bytes=56 << 20,
        ),
    )(x_p, x_p, means, alpha)

    return out.reshape(n_pad)[:n]


# speedup vs baseline: 1.2695x; 1.2695x over previous
import jax
import jax.numpy as jnp
from jax.experimental import pallas as pl
from jax.experimental.pallas import tpu as pltpu

_LANE = 128
_TN = 4096


def _round_up(v, m):
    return (v + m - 1) // m * m


def _body(x, m2, abb, t4):
    f32 = jnp.float32
    xb = x.astype(jnp.bfloat16)
    term23 = jnp.dot(xb, m2, preferred_element_type=f32)
    xa = jnp.dot(xb, abb, preferred_element_type=f32)
    term1 = jnp.sum(xa * x, axis=1, keepdims=True)
    qmin = term1 + jnp.min(t4 - term23, axis=1, keepdims=True)
    packed = jnp.broadcast_to(qmin, (qmin.shape[0], _LANE))
    return packed.T[:1, :]


def _maha_kernel(x0_ref, x1_ref, means_ref, alpha_ref, o0_ref, o1_ref):
    bf16 = jnp.bfloat16
    f32 = jnp.float32
    mb = means_ref[...]
    abb = alpha_ref[...].astype(bf16)
    mbb = mb.astype(bf16)
    am = jnp.dot(abb, mbb, preferred_element_type=f32)
    atm = jax.lax.dot_general(abb, mbb, (((0,), (0,)), ((), ())),
                              preferred_element_type=f32)
    t4 = jnp.sum(mb * am, axis=0, keepdims=True)
    m2 = (am + atm).astype(bf16)

    o0_ref[...] = _body(x0_ref[...], m2, abb, t4)
    o1_ref[...] = _body(x1_ref[...], m2, abb, t4)


def kernel(x, means, alpha):
    n, d = x.shape
    d_m, c = means.shape
    assert d == d_m and alpha.shape == (d, d)

    f32 = jnp.float32
    x = x.astype(f32)
    means = means.astype(f32)
    alpha = alpha.astype(f32)

    tn = min(_TN, _round_up(n, _LANE))
    n_pad = _round_up(n, 2 * tn)
    num_pairs = n_pad // (2 * tn)
    x_p = x if n_pad == n else jnp.zeros((n_pad, d), f32).at[:n, :].set(x)

    o0, o1 = pl.pallas_call(
        _maha_kernel,
        out_shape=(jax.ShapeDtypeStruct((num_pairs, 1, tn), f32),
                   jax.ShapeDtypeStruct((num_pairs, 1, tn), f32)),
        grid=(num_pairs,),
        in_specs=[
            pl.BlockSpec((tn, d), lambda i: (2 * i, 0)),
            pl.BlockSpec((tn, d), lambda i: (2 * i + 1, 0)),
            pl.BlockSpec((d, c), lambda i: (0, 0),
                         pipeline_mode=pl.Buffered(1)),
            pl.BlockSpec((d, d), lambda i: (0, 0),
                         pipeline_mode=pl.Buffered(1)),
        ],
        out_specs=(pl.BlockSpec((None, 1, tn), lambda i: (i, 0, 0)),
                   pl.BlockSpec((None, 1, tn), lambda i: (i, 0, 0))),
        compiler_params=pltpu.CompilerParams(
            dimension_semantics=("parallel",),
            vmem_limit_bytes=56 << 20,
        ),
    )(x_p, x_p, means, alpha)

    out = jnp.concatenate([o0, o1], axis=1)
    return out.reshape(n_pad)[:n]


# speedup vs baseline: 1.4478x; 1.1404x over previous
import jax
import jax.numpy as jnp
from jax.experimental import pallas as pl
from jax.experimental.pallas import tpu as pltpu

_LANE = 128
_TN = 8192


def _round_up(v, m):
    return (v + m - 1) // m * m


def _maha_kernel(x_ref, means_ref, alpha_ref, out_ref,
                 abb_ref, m2_ref, t4_ref):
    bf16 = jnp.bfloat16
    f32 = jnp.float32

    @pl.when(pl.program_id(0) == 0)
    def _prep():
        mb = means_ref[...]
        abb = alpha_ref[...].astype(bf16)
        mbb = mb.astype(bf16)
        am = jnp.dot(abb, mbb, preferred_element_type=f32)
        atm = jax.lax.dot_general(abb, mbb, (((0,), (0,)), ((), ())),
                                  preferred_element_type=f32)
        abb_ref[...] = abb
        t4_ref[...] = jnp.sum(mb * am, axis=0, keepdims=True)
        m2_ref[...] = (am + atm).astype(bf16)

    x = x_ref[...]
    xb = x.astype(bf16)
    term23 = jnp.dot(xb, m2_ref[...], preferred_element_type=f32)
    xa = jnp.dot(xb, abb_ref[...], preferred_element_type=f32)
    term1 = jnp.sum(xa * x, axis=1, keepdims=True)
    qmin = term1 + jnp.min(t4_ref[...] - term23, axis=1, keepdims=True)
    packed = jnp.broadcast_to(qmin, (qmin.shape[0], _LANE))
    out_ref[...] = packed.T[:1, :]


def kernel(x, means, alpha):
    n, d = x.shape
    d_m, c = means.shape
    assert d == d_m and alpha.shape == (d, d)

    f32 = jnp.float32
    x = x.astype(f32)
    means = means.astype(f32)
    alpha = alpha.astype(f32)

    tn = min(_TN, _round_up(n, _LANE))
    n_pad = _round_up(n, tn)
    num_tiles = n_pad // tn
    x_p = x if n_pad == n else jnp.zeros((n_pad, d), f32).at[:n, :].set(x)

    out = pl.pallas_call(
        _maha_kernel,
        out_shape=jax.ShapeDtypeStruct((num_tiles, 1, tn), f32),
        grid=(num_tiles,),
        in_specs=[
            pl.BlockSpec((tn, d), lambda i: (i, 0)),
            pl.BlockSpec((d, c), lambda i: (0, 0),
                         pipeline_mode=pl.Buffered(1)),
            pl.BlockSpec((d, d), lambda i: (0, 0),
                         pipeline_mode=pl.Buffered(1)),
        ],
        out_specs=pl.BlockSpec((None, 1, tn), lambda i: (i, 0, 0)),
        scratch_shapes=[
            pltpu.VMEM((d, d), jnp.bfloat16),
            pltpu.VMEM((d, c), jnp.bfloat16),
            pltpu.VMEM((1, c), f32),
        ],
        compiler_params=pltpu.CompilerParams(
            dimension_semantics=("arbitrary",),
            vmem_limit_bytes=56 << 20,
        ),
    )(x_p, means, alpha)

    return out.reshape(n_pad)[:n]
